# Initial kernel scaffold; baseline (speedup 1.0000x reference)
import jax
import jax.numpy as jnp
from jax.experimental import pallas as pl


def kernel(x, edge_index, W1_0, g1_0, b1_0, W2_0, W1_1, g1_1, b1_1, W2_1, og0, ob0, og1, ob1, lpW0, lpb0, lpW1, lpb1, lpW2, lpb2):
    raise NotImplementedError("write your pallas kernel here")


# speedup vs baseline: 2.7705x; 2.7705x over previous
import functools

import jax
import jax.numpy as jnp
from jax import lax
from jax.experimental import pallas as pl
from jax.experimental.pallas import tpu as pltpu
from jax.experimental.pallas import tpu_sc as plsc

N = 10000
E = 320000
D = 128
H = 128
O = 64

NUM_CORES = 2
NUM_SUBCORES = 16
NUM_TILES = NUM_CORES * NUM_SUBCORES
CHUNK = 128
CHUNKS = 80
EP = NUM_TILES * CHUNKS * CHUNK
ACC_ROWS = 10240
ZERO_ROWS = ACC_ROWS // NUM_SUBCORES

@functools.cache
def _get_sc_agg():
    mesh = plsc.VectorSubcoreMesh(core_axis_name="c", subcore_axis_name="s")

    @functools.partial(
        pl.kernel,
        out_type=jax.ShapeDtypeStruct((NUM_CORES, ACC_ROWS, D), jnp.float32),
        mesh=mesh,
        scratch_types=[
            pltpu.VMEM((CHUNKS, CHUNK), jnp.int32),
            pltpu.VMEM((CHUNKS, CHUNK), jnp.int32),
            pltpu.VMEM((CHUNK, D), jnp.float32),
            pltpu.VMEM_SHARED((ACC_ROWS, D), jnp.float32),
            pltpu.SemaphoreType.DMA,
        ],
    )
    def _sc_agg(h_hbm, src_hbm, dst_hbm, zeros_hbm, out_hbm,
                src_v, dst_v, rows_v, acc_sh, sem):
        c = lax.axis_index("c")
        s = lax.axis_index("s")
        wid = c * NUM_SUBCORES + s
        for i in range(ZERO_ROWS // 128):
            pltpu.sync_copy(zeros_hbm, acc_sh.at[pl.ds(s * ZERO_ROWS + i * 128, 128)])
        pltpu.sync_copy(src_hbm.at[wid], src_v)
        pltpu.sync_copy(dst_hbm.at[wid], dst_v)
        plsc.subcore_barrier()

        def body(i, _):
            pltpu.async_copy(h_hbm.at[src_v.at[i]], rows_v, sem).wait()
            pltpu.sync_copy(rows_v, acc_sh.at[dst_v.at[i]], add=True)
            return ()

        lax.fori_loop(0, CHUNKS, body, ())
        plsc.subcore_barrier()
        pltpu.sync_copy(acc_sh.at[pl.ds(s * ZERO_ROWS, ZERO_ROWS)],
                        out_hbm.at[c, pl.ds(s * ZERO_ROWS, ZERO_ROWS)])

    return _sc_agg


def _bn_relu(y, g, b, eps=1e-5):
    mu = jnp.mean(y, axis=0, keepdims=True)
    var = jnp.mean((y - mu) * (y - mu), axis=0, keepdims=True)
    return jnp.maximum((y - mu) * lax.rsqrt(var + eps) * g + b, 0.0)


def _mm(a, w):
    return lax.dot_general(a, w, (((1,), (1,)), ((), ())),
                           preferred_element_type=jnp.float32,
                           precision=lax.Precision.HIGHEST)


def _dense0_body(x_ref, agg_ref, W1_ref, g1_ref, b1_ref, W2_ref,
                 og_ref, ob_ref, lpW0_ref, lpb0_ref, lpW1_ref, lpb1_ref,
                 h1_ref, score_ref):
    x = x_ref[...]
    hin = x + agg_ref[0, :N] + agg_ref[1, :N]
    y = _mm(hin, W1_ref[...])
    y = _bn_relu(y, g1_ref[...], b1_ref[...])
    z = _mm(y, W2_ref[...])
    h1 = _bn_relu(z, og_ref[...], ob_ref[...])
    h1_ref[...] = h1
    score_ref[...] = (_mm(x, lpW0_ref[...]) + lpb0_ref[...]
                      + _mm(h1, lpW1_ref[...]) + lpb1_ref[...])


def _dense1_body(h1_ref, agg_ref, W1_ref, g1_ref, b1_ref, W2_ref,
                 og_ref, ob_ref, lpW2_ref, lpb2_ref, sp_ref, score_ref):
    h1 = h1_ref[...]
    hin = h1 + agg_ref[0, :N] + agg_ref[1, :N]
    y = _mm(hin, W1_ref[...])
    y = _bn_relu(y, g1_ref[...], b1_ref[...])
    z = _mm(y, W2_ref[...])
    h2 = _bn_relu(z, og_ref[...], ob_ref[...])
    score_ref[...] = sp_ref[...] + _mm(h2, lpW2_ref[...]) + lpb2_ref[...]


_dense0 = pl.pallas_call(
    _dense0_body,
    out_shape=(jax.ShapeDtypeStruct((N, H), jnp.float32),
               jax.ShapeDtypeStruct((N, O), jnp.float32)),
)

_dense1 = pl.pallas_call(
    _dense1_body,
    out_shape=jax.ShapeDtypeStruct((N, O), jnp.float32),
)


def kernel(x, edge_index, W1_0, g1_0, b1_0, W2_0, W1_1, g1_1, b1_1, W2_1,
           og0, ob0, og1, ob1, lpW0, lpb0, lpW1, lpb1, lpW2, lpb2):
    src = edge_index[0]
    dst = edge_index[1]
    pad = EP - E
    src_r = jnp.concatenate([src, jnp.zeros((pad,), jnp.int32)]).reshape(
        NUM_TILES, CHUNKS, CHUNK)
    dst_r = jnp.concatenate([dst, jnp.full((pad,), N, jnp.int32)]).reshape(
        NUM_TILES, CHUNKS, CHUNK)
    zeros128 = jnp.zeros((128, D), jnp.float32)

    g1_0r, b1_0r = g1_0.reshape(1, H), b1_0.reshape(1, H)
    g1_1r, b1_1r = g1_1.reshape(1, H), b1_1.reshape(1, H)
    og0r, ob0r = og0.reshape(1, H), ob0.reshape(1, H)
    og1r, ob1r = og1.reshape(1, H), ob1.reshape(1, H)
    lpb0r = lpb0.reshape(1, O)
    lpb1r = lpb1.reshape(1, O)
    lpb2r = lpb2.reshape(1, O)

    sc_agg = _get_sc_agg()
    agg0 = sc_agg(x, src_r, dst_r, zeros128)
    h1, score_part = _dense0(x, agg0, W1_0, g1_0r, b1_0r, W2_0,
                             og0r, ob0r, lpW0, lpb0r, lpW1, lpb1r)
    agg1 = sc_agg(h1, src_r, dst_r, zeros128)
    score = _dense1(h1, agg1, W1_1, g1_1r, b1_1r, W2_1,
                    og1r, ob1r, lpW2, lpb2r, score_part)
    return score


# speedup vs baseline: 3.0694x; 1.1079x over previous
import functools

import jax
import jax.numpy as jnp
from jax import lax
from jax.experimental import pallas as pl
from jax.experimental.pallas import tpu as pltpu
from jax.experimental.pallas import tpu_sc as plsc

N = 10000
E = 320000
D = 128
H = 128
O = 64

NUM_CORES = 2
NUM_SUBCORES = 16
NUM_TILES = NUM_CORES * NUM_SUBCORES
CHUNK = 128
CHUNKS = 80
EP = NUM_TILES * CHUNKS * CHUNK
ACC_ROWS = 10240
ZERO_ROWS = ACC_ROWS // NUM_SUBCORES

@functools.cache
def _get_sc_agg():
    mesh = plsc.VectorSubcoreMesh(core_axis_name="c", subcore_axis_name="s")

    @functools.partial(
        pl.kernel,
        out_type=jax.ShapeDtypeStruct((NUM_CORES, ACC_ROWS, D), jnp.float32),
        mesh=mesh,
        scratch_types=[
            pltpu.VMEM((CHUNKS // 2, CHUNK), jnp.int32),
            pltpu.VMEM((CHUNKS // 2, CHUNK), jnp.int32),
            pltpu.VMEM((2, CHUNK, D), jnp.float32),
            pltpu.VMEM_SHARED((ACC_ROWS, D), jnp.float32),
            pltpu.SemaphoreType.DMA,
            pltpu.SemaphoreType.DMA,
        ],
    )
    def _sc_agg(h_hbm, src_hbm, dst_hbm, zeros_hbm, out_hbm,
                src_v, dst_v, rows_v, acc_sh, sem0, sem1):
        c = lax.axis_index("c")
        s = lax.axis_index("s")
        wid = c * NUM_SUBCORES + s
        half = CHUNKS // 2
        for i in range(ZERO_ROWS // 128):
            pltpu.sync_copy(zeros_hbm, acc_sh.at[pl.ds(s * ZERO_ROWS + i * 128, 128)])
        plsc.subcore_barrier()

        sems = (sem0, sem1)
        for ph in range(2):
            pltpu.sync_copy(src_hbm.at[wid, pl.ds(ph * half, half)], src_v)
            pltpu.sync_copy(dst_hbm.at[wid, pl.ds(ph * half, half)], dst_v)
            pltpu.async_copy(h_hbm.at[src_v.at[0]], rows_v.at[0], sem0)
            pltpu.async_copy(h_hbm.at[src_v.at[1]], rows_v.at[1], sem1)

            def outer(t, _):
                j = t * 2
                for b in range(2):
                    i = j + b
                    pltpu.make_async_copy(h_hbm.at[src_v.at[i]], rows_v.at[b],
                                          sems[b]).wait()
                    pltpu.sync_copy(rows_v.at[b], acc_sh.at[dst_v.at[i]],
                                    add=True)

                    @pl.when(i + 2 < half)
                    def _():
                        pltpu.async_copy(h_hbm.at[src_v.at[i + 2]],
                                         rows_v.at[b], sems[b])
                return ()

            lax.fori_loop(0, half // 2, outer, ())
        plsc.subcore_barrier()
        pltpu.sync_copy(acc_sh.at[pl.ds(s * ZERO_ROWS, ZERO_ROWS)],
                        out_hbm.at[c, pl.ds(s * ZERO_ROWS, ZERO_ROWS)])

    return _sc_agg


def _bn_relu(y, g, b, eps=1e-5):
    mu = jnp.mean(y, axis=0, keepdims=True)
    var = jnp.mean((y - mu) * (y - mu), axis=0, keepdims=True)
    return jnp.maximum((y - mu) * lax.rsqrt(var + eps) * g + b, 0.0)


def _mm(a, w):
    return lax.dot_general(a, w, (((1,), (1,)), ((), ())),
                           preferred_element_type=jnp.float32,
                           precision=lax.Precision.HIGHEST)


def _dense0_body(x_ref, agg_ref, W1_ref, g1_ref, b1_ref, W2_ref,
                 og_ref, ob_ref, lpW0_ref, lpb0_ref, lpW1_ref, lpb1_ref,
                 h1_ref, score_ref):
    x = x_ref[...]
    hin = x + agg_ref[0, :N] + agg_ref[1, :N]
    y = _mm(hin, W1_ref[...])
    y = _bn_relu(y, g1_ref[...], b1_ref[...])
    z = _mm(y, W2_ref[...])
    h1 = _bn_relu(z, og_ref[...], ob_ref[...])
    h1_ref[...] = h1
    score_ref[...] = (_mm(x, lpW0_ref[...]) + lpb0_ref[...]
                      + _mm(h1, lpW1_ref[...]) + lpb1_ref[...])


def _dense1_body(h1_ref, agg_ref, W1_ref, g1_ref, b1_ref, W2_ref,
                 og_ref, ob_ref, lpW2_ref, lpb2_ref, sp_ref, score_ref):
    h1 = h1_ref[...]
    hin = h1 + agg_ref[0, :N] + agg_ref[1, :N]
    y = _mm(hin, W1_ref[...])
    y = _bn_relu(y, g1_ref[...], b1_ref[...])
    z = _mm(y, W2_ref[...])
    h2 = _bn_relu(z, og_ref[...], ob_ref[...])
    score_ref[...] = sp_ref[...] + _mm(h2, lpW2_ref[...]) + lpb2_ref[...]


_dense0 = pl.pallas_call(
    _dense0_body,
    out_shape=(jax.ShapeDtypeStruct((N, H), jnp.float32),
               jax.ShapeDtypeStruct((N, O), jnp.float32)),
)

_dense1 = pl.pallas_call(
    _dense1_body,
    out_shape=jax.ShapeDtypeStruct((N, O), jnp.float32),
)


def kernel(x, edge_index, W1_0, g1_0, b1_0, W2_0, W1_1, g1_1, b1_1, W2_1,
           og0, ob0, og1, ob1, lpW0, lpb0, lpW1, lpb1, lpW2, lpb2):
    src = edge_index[0]
    dst = edge_index[1]
    pad = EP - E
    src_r = jnp.concatenate([src, jnp.zeros((pad,), jnp.int32)]).reshape(
        NUM_TILES, CHUNKS, CHUNK)
    dst_r = jnp.concatenate([dst, jnp.full((pad,), N, jnp.int32)]).reshape(
        NUM_TILES, CHUNKS, CHUNK)
    zeros128 = jnp.zeros((128, D), jnp.float32)

    g1_0r, b1_0r = g1_0.reshape(1, H), b1_0.reshape(1, H)
    g1_1r, b1_1r = g1_1.reshape(1, H), b1_1.reshape(1, H)
    og0r, ob0r = og0.reshape(1, H), ob0.reshape(1, H)
    og1r, ob1r = og1.reshape(1, H), ob1.reshape(1, H)
    lpb0r = lpb0.reshape(1, O)
    lpb1r = lpb1.reshape(1, O)
    lpb2r = lpb2.reshape(1, O)

    sc_agg = _get_sc_agg()
    agg0 = sc_agg(x, src_r, dst_r, zeros128)
    h1, score_part = _dense0(x, agg0, W1_0, g1_0r, b1_0r, W2_0,
                             og0r, ob0r, lpW0, lpb0r, lpW1, lpb1r)
    agg1 = sc_agg(h1, src_r, dst_r, zeros128)
    score = _dense1(h1, agg1, W1_1, g1_1r, b1_1r, W2_1,
                    og1r, ob1r, lpW2, lpb2r, score_part)
    return score
